# Initial kernel scaffold; baseline (speedup 1.0000x reference)
import jax
import jax.numpy as jnp
from jax.experimental import pallas as pl


def kernel(left_features, edge_indices, edge_features, right_features, scatter_out_size, W_l, b_l, W_e, W_r, gamma1, beta1, W_f, b_f, gamma2, beta2, W_o1, b_o1, W_o2, b_o2):
    raise NotImplementedError("write your pallas kernel here")


# speedup vs baseline: 1.8541x; 1.8541x over previous
import functools

import jax
import jax.numpy as jnp
from jax import lax
from jax.experimental import pallas as pl
from jax.experimental.pallas import tpu as pltpu
from jax.experimental.pallas import tpu_sc as plsc

EMB = 128
NC = 2
NS = 16
NW = NC * NS
LANES = 16
CH = 80
CNT_W = 16
ZCH = 80
_SC_PARAMS = pltpu.CompilerParams(use_tc_tiling_on_sc=False)


def _dotT(x, w):
    return lax.dot_general(x, w, (((1,), (1,)), ((), ())),
                           preferred_element_type=jnp.float32)


def _lr_body(lf_ref, rf_ref, wl_ref, bl_ref, wr_ref, l_ref, r_ref):
    l_ref[...] = _dotT(lf_ref[...], wl_ref[...]) + bl_ref[...]
    r_ref[...] = _dotT(rf_ref[...], wr_ref[...])


def _tc_lr(lf, rf, W_l, b_l, W_r):
    n = lf.shape[0]
    blk = 2000
    grid = (n // blk,)
    return pl.pallas_call(
        _lr_body,
        grid=grid,
        in_specs=[
            pl.BlockSpec((blk, EMB), lambda i: (i, 0)),
            pl.BlockSpec((blk, EMB), lambda i: (i, 0)),
            pl.BlockSpec((EMB, EMB), lambda i: (0, 0)),
            pl.BlockSpec((1, EMB), lambda i: (0, 0)),
            pl.BlockSpec((EMB, EMB), lambda i: (0, 0)),
        ],
        out_specs=[
            pl.BlockSpec((blk, EMB), lambda i: (i, 0)),
            pl.BlockSpec((blk, EMB), lambda i: (i, 0)),
        ],
        out_shape=[jax.ShapeDtypeStruct((n, EMB), jnp.float32)] * 2,
    )(lf, rf, W_l, b_l.reshape(1, EMB), W_r)


def _sc_stats_body(n_edges, l_hbm, r_hbm, src_hbm, dst_hbm, ef_hbm, w_hbm,
                   osum_hbm, osq_hbm,
                   src_v, dst_v, ef_v, lrows, rrows, w_v, sum_v, sq_v):
    cid = lax.axis_index("c")
    sid = lax.axis_index("s")
    wid = sid * NC + cid
    ept = n_edges // NW
    nch = ept // CH

    pltpu.sync_copy(w_hbm, w_v)
    wg = [w_v[pl.ds(16 * g, 16)] for g in range(8)]
    zero = jnp.zeros((16,), jnp.float32)
    for g in range(8):
        sum_v[pl.ds(16 * g, 16)] = zero
        sq_v[pl.ds(16 * g, 16)] = zero

    def chunk(ci, carry):
        base = wid * ept + ci * CH
        pltpu.sync_copy(src_hbm.at[pl.ds(base, CH)], src_v)
        pltpu.sync_copy(dst_hbm.at[pl.ds(base, CH)], dst_v)
        pltpu.sync_copy(ef_hbm.at[pl.ds(base, CH)], ef_v)
        pltpu.sync_copy(l_hbm.at[src_v], lrows)
        pltpu.sync_copy(r_hbm.at[dst_v], rrows)
        def egroup(eg, sq):
            s, q = sq
            e0 = eg * 16
            ef16 = ef_v[pl.ds(e0, 16)]
            for i in range(16):
                efb = jnp.full((16,), ef16[i], jnp.float32)
                for g in range(8):
                    j = lrows[e0 + i, pl.ds(16 * g, 16)] \
                        + rrows[e0 + i, pl.ds(16 * g, 16)] + efb * wg[g]
                    s = s[:g] + (s[g] + j,) + s[g + 1:]
                    q = q[:g] + (q[g] + j * j,) + q[g + 1:]
            return (s, q)

        s, q = lax.fori_loop(0, CH // 16, egroup,
                             ((zero,) * 8, (zero,) * 8))
        for g in range(8):
            sum_v[pl.ds(16 * g, 16)] += s[g]
            sq_v[pl.ds(16 * g, 16)] += q[g]
        return carry

    lax.fori_loop(0, nch, chunk, 0)
    pltpu.sync_copy(sum_v, osum_hbm.at[wid])
    pltpu.sync_copy(sq_v, osq_hbm.at[wid])


def _sc_stats(L, R, src, dst, ef, wvec):
    n_edges = src.shape[0]
    mesh = plsc.VectorSubcoreMesh(core_axis_name="c", subcore_axis_name="s")
    return pl.kernel(
        functools.partial(_sc_stats_body, n_edges),
        mesh=mesh,
        compiler_params=_SC_PARAMS,
        out_type=[jax.ShapeDtypeStruct((NW, EMB), jnp.float32)] * 2,
        scratch_types=[
            pltpu.VMEM((CH,), jnp.int32),
            pltpu.VMEM((CH,), jnp.int32),
            pltpu.VMEM((CH,), jnp.float32),
            pltpu.VMEM((CH, EMB), jnp.float32),
            pltpu.VMEM((CH, EMB), jnp.float32),
            pltpu.VMEM((EMB,), jnp.float32),
            pltpu.VMEM((EMB,), jnp.float32),
            pltpu.VMEM((EMB,), jnp.float32),
        ],
    )(L, R, src, dst, ef, wvec)


def _sc_scatter_body(n_edges, n_right,
                     l_hbm, r_hbm, src_hbm, dst_hbm, ef_hbm, w_hbm,
                     scale_hbm, shift_hbm, out_hbm, ocnt_hbm,
                     src_v, dst_v, ef_v, lrows, rrows,
                     w_v, scale_v, shift_v, joint_v, ones_v,
                     acc_sh, cnt_sh):
    cid = lax.axis_index("c")
    sid = lax.axis_index("s")
    wid = sid * NC + cid
    ept = n_edges // NW
    nch = ept // CH
    nz = n_right // ZCH
    nzt = (nz + NS - 1) // NS

    zero = jnp.zeros((16,), jnp.float32)
    onecol = jnp.where(lax.iota(jnp.int32, 16) == 0,
                       jnp.float32(1.0), jnp.float32(0.0))

    def zrow(r, carry):
        for g in range(EMB // 16):
            lrows[r, pl.ds(16 * g, 16)] = zero
        ones_v[r, :] = zero
        return carry
    lax.fori_loop(0, ZCH, zrow, 0)
    for t in range(nzt):
        k = sid + NS * t

        @pl.when(k < nz)
        def _():
            r0 = pl.multiple_of(k * ZCH, 8)
            pltpu.sync_copy(lrows.at[pl.ds(0, ZCH), :],
                            acc_sh.at[pl.ds(r0, ZCH), :])
            pltpu.sync_copy(ones_v, cnt_sh.at[pl.ds(r0, ZCH), :])
    plsc.subcore_barrier()

    pltpu.sync_copy(w_hbm, w_v)
    pltpu.sync_copy(scale_hbm, scale_v)
    pltpu.sync_copy(shift_hbm, shift_v)
    wg = [w_v[pl.ds(16 * g, 16)] for g in range(8)]
    sg = [scale_v[pl.ds(16 * g, 16)] for g in range(8)]
    tg = [shift_v[pl.ds(16 * g, 16)] for g in range(8)]

    def orow(r, carry):
        ones_v[r, :] = onecol
        return carry
    lax.fori_loop(0, CH, orow, 0)

    def chunk(ci, carry):
        base = wid * ept + ci * CH
        pltpu.sync_copy(src_hbm.at[pl.ds(base, CH)], src_v)
        pltpu.sync_copy(dst_hbm.at[pl.ds(base, CH)], dst_v)
        pltpu.sync_copy(ef_hbm.at[pl.ds(base, CH)], ef_v)
        pltpu.sync_copy(l_hbm.at[src_v], lrows)
        pltpu.sync_copy(r_hbm.at[dst_v], rrows)
        def egroup(eg, c):
            e0 = eg * 16
            ef16 = ef_v[pl.ds(e0, 16)]
            for i in range(16):
                efb = jnp.full((16,), ef16[i], jnp.float32)
                for g in range(8):
                    x = lrows[e0 + i, pl.ds(16 * g, 16)] \
                        + rrows[e0 + i, pl.ds(16 * g, 16)] + efb * wg[g]
                    joint_v[e0 + i, pl.ds(16 * g, 16)] = jnp.maximum(
                        x * sg[g] + tg[g], 0.0)
            return c

        lax.fori_loop(0, CH // 16, egroup, 0)
        pltpu.sync_copy(joint_v, acc_sh.at[dst_v], add=True)
        pltpu.sync_copy(ones_v, cnt_sh.at[dst_v], add=True)
        return carry

    lax.fori_loop(0, nch, chunk, 0)
    plsc.subcore_barrier()

    for t in range(nzt):
        k = sid + NS * t

        @pl.when(k < nz)
        def _():
            r0 = pl.multiple_of(k * ZCH, 8)
            pltpu.sync_copy(acc_sh.at[pl.ds(r0, ZCH), :],
                            out_hbm.at[cid, pl.ds(r0, ZCH), :])
            pltpu.sync_copy(cnt_sh.at[pl.ds(r0, ZCH), :],
                            ocnt_hbm.at[cid, pl.ds(r0, ZCH), :])


def _sc_scatter(L, R, src, dst, ef, wvec, scale, shift):
    n_edges = src.shape[0]
    n_right = R.shape[0]
    mesh = plsc.VectorSubcoreMesh(core_axis_name="c", subcore_axis_name="s")
    return pl.kernel(
        functools.partial(_sc_scatter_body, n_edges, n_right),
        mesh=mesh,
        compiler_params=_SC_PARAMS,
        out_type=[jax.ShapeDtypeStruct((NC, n_right, EMB), jnp.float32),
                  jax.ShapeDtypeStruct((NC, n_right, CNT_W), jnp.float32)],
        scratch_types=[
            pltpu.VMEM((CH,), jnp.int32),
            pltpu.VMEM((CH,), jnp.int32),
            pltpu.VMEM((CH,), jnp.float32),
            pltpu.VMEM((CH, EMB), jnp.float32),
            pltpu.VMEM((CH, EMB), jnp.float32),
            pltpu.VMEM((EMB,), jnp.float32),
            pltpu.VMEM((EMB,), jnp.float32),
            pltpu.VMEM((EMB,), jnp.float32),
            pltpu.VMEM((CH, EMB), jnp.float32),
            pltpu.VMEM((CH, CNT_W), jnp.float32),
            pltpu.VMEM_SHARED((n_right, EMB), jnp.float32),
            pltpu.VMEM_SHARED((n_right, CNT_W), jnp.float32),
        ],
    )(L, R, src, dst, ef, wvec, scale, shift)


def _tail_body(acc_ref, cnt_ref, rf_ref, wf_ref, bf_ref, g2_ref, b2_ref,
               wo1a_ref, wo1b_ref, bo1_ref, wo2_ref, bo2_ref, out_ref):
    feat = acc_ref[0] + acc_ref[1]
    cnt = (cnt_ref[0] + cnt_ref[1])[:, 0:1]
    conv = _dotT(feat, wf_ref[...]) + cnt * bf_ref[...]
    mu = jnp.mean(conv, axis=0, keepdims=True)
    var = jnp.mean((conv - mu) ** 2, axis=0, keepdims=True)
    convn = g2_ref[...] * (conv - mu) / jnp.sqrt(var + 1e-5) + b2_ref[...]
    h = jnp.maximum(
        _dotT(convn, wo1a_ref[...]) + _dotT(rf_ref[...], wo1b_ref[...])
        + bo1_ref[...], 0.0)
    out_ref[...] = jnp.maximum(_dotT(h, wo2_ref[...]) + bo2_ref[...], 0.0)


def _tc_tail(acc, cnt, rf, W_f, b_f, gamma2, beta2, W_o1, b_o1, W_o2, b_o2):
    n = rf.shape[0]
    full2 = pl.BlockSpec((EMB, EMB), lambda: (0, 0))
    row = pl.BlockSpec((1, EMB), lambda: (0, 0))
    return pl.pallas_call(
        _tail_body,
        in_specs=[
            pl.BlockSpec((NC, n, EMB), lambda: (0, 0, 0)),
            pl.BlockSpec((NC, n, CNT_W), lambda: (0, 0, 0)),
            pl.BlockSpec((n, EMB), lambda: (0, 0)),
            full2, row, row, row, full2, full2, row, full2, row,
        ],
        out_specs=pl.BlockSpec((n, EMB), lambda: (0, 0)),
        out_shape=jax.ShapeDtypeStruct((n, EMB), jnp.float32),
    )(acc, cnt, rf, W_f, b_f.reshape(1, EMB), gamma2.reshape(1, EMB),
      beta2.reshape(1, EMB), W_o1[:, :EMB], W_o1[:, EMB:],
      b_o1.reshape(1, EMB), W_o2, b_o2.reshape(1, EMB))


def kernel(left_features, edge_indices, edge_features, right_features,
           scatter_out_size, W_l, b_l, W_e, W_r, gamma1, beta1,
           W_f, b_f, gamma2, beta2, W_o1, b_o1, W_o2, b_o2):
    n_edges = edge_indices.shape[1]
    src = edge_indices[0].astype(jnp.int32)
    dst = edge_indices[1].astype(jnp.int32)
    ef = edge_features[:, 0].astype(jnp.float32)
    wvec = W_e[:, 0].astype(jnp.float32)

    L, R = _tc_lr(left_features, right_features, W_l, b_l, W_r)

    psum, psq = _sc_stats(L, R, src, dst, ef, wvec)
    s1 = jnp.sum(psum, axis=0)
    s2 = jnp.sum(psq, axis=0)
    mu = s1 / n_edges
    var = s2 / n_edges - mu * mu
    inv = 1.0 / jnp.sqrt(var + 1e-5)
    scale = gamma1 * inv
    shift = beta1 - mu * scale

    acc, cnt = _sc_scatter(L, R, src, dst, ef, wvec, scale, shift)

    return _tc_tail(acc, cnt, right_features, W_f, b_f, gamma2, beta2,
                    W_o1, b_o1, W_o2, b_o2)


# speedup vs baseline: 2.3830x; 1.2853x over previous
import functools

import jax
import jax.numpy as jnp
from jax import lax
from jax.experimental import pallas as pl
from jax.experimental.pallas import tpu as pltpu
from jax.experimental.pallas import tpu_sc as plsc

EMB = 128
NG = EMB // 16
NC = 2
NS = 16
NW = NC * NS
CH = 40
SBC = 50
ACC_W = EMB + 16
ZROWS = CH
_SC_PARAMS = pltpu.CompilerParams(use_tc_tiling_on_sc=False)

_NFULL = CH // 16
_TAIL = CH % 16


def _dotT(x, w):
    return lax.dot_general(x, w, (((1,), (1,)), ((), ())),
                           preferred_element_type=jnp.float32)


def _lr_body(lf_ref, rf_ref, wl_ref, bl_ref, wr_ref, l_ref, r_ref):
    l_ref[...] = _dotT(lf_ref[...], wl_ref[...]) + bl_ref[...]
    r_ref[...] = _dotT(rf_ref[...], wr_ref[...])


def _tc_lr(lf, rf, W_l, b_l, W_r):
    n = lf.shape[0]
    blk = 2000
    grid = (n // blk,)
    return pl.pallas_call(
        _lr_body,
        grid=grid,
        in_specs=[
            pl.BlockSpec((blk, EMB), lambda i: (i, 0)),
            pl.BlockSpec((blk, EMB), lambda i: (i, 0)),
            pl.BlockSpec((EMB, EMB), lambda i: (0, 0)),
            pl.BlockSpec((1, EMB), lambda i: (0, 0)),
            pl.BlockSpec((EMB, EMB), lambda i: (0, 0)),
        ],
        out_specs=[
            pl.BlockSpec((blk, EMB), lambda i: (i, 0)),
            pl.BlockSpec((blk, EMB), lambda i: (i, 0)),
        ],
        out_shape=[jax.ShapeDtypeStruct((n, EMB), jnp.float32)] * 2,
    )(lf, rf, W_l, b_l.reshape(1, EMB), W_r)


def _drain_gather(l_hbm, r_hbm, src_sb, dst_sb, lbuf, rbuf, sem):
    pltpu.make_async_copy(l_hbm.at[src_sb.at[0]], lbuf, sem).wait()
    pltpu.make_async_copy(r_hbm.at[dst_sb.at[0]], rbuf, sem).wait()


def _issue_gather(l_hbm, r_hbm, src_sb, dst_sb, cc, lbuf, rbuf, sem):
    pltpu.async_copy(l_hbm.at[src_sb.at[cc]], lbuf, sem)
    pltpu.async_copy(r_hbm.at[dst_sb.at[cc]], rbuf, sem)


def _sc_stats_body(n_edges, l_hbm, r_hbm, src_hbm, dst_hbm, ef_hbm, w_hbm,
                   osum_hbm, osq_hbm,
                   src_sb, dst_sb, ef_sb, l0, r0, l1, r1,
                   w_v, sum_v, sq_v, sidx, sg0, sg1):
    cid = lax.axis_index("c")
    sid = lax.axis_index("s")
    wid = sid * NC + cid
    cpt = n_edges // NW // CH
    nsb = cpt // SBC
    row_base = wid * cpt

    pltpu.sync_copy(w_hbm, w_v)
    wg = [w_v[pl.ds(16 * g, 16)] for g in range(NG)]
    zero = jnp.zeros((16,), jnp.float32)
    for g in range(NG):
        sum_v[pl.ds(16 * g, 16)] = zero
        sq_v[pl.ds(16 * g, 16)] = zero

    lrows = [l0, l1]
    rrows = [r0, r1]
    sg = [sg0, sg1]

    def superblock(sb, carry):
        r0_ = row_base + sb * SBC
        pltpu.async_copy(src_hbm.at[pl.ds(r0_, SBC), :], src_sb, sidx)
        pltpu.async_copy(dst_hbm.at[pl.ds(r0_, SBC), :], dst_sb, sidx)
        pltpu.async_copy(ef_hbm.at[pl.ds(r0_, SBC), :], ef_sb, sidx)
        pltpu.make_async_copy(src_hbm.at[pl.ds(0, SBC), :], src_sb, sidx).wait()
        pltpu.make_async_copy(dst_hbm.at[pl.ds(0, SBC), :], dst_sb, sidx).wait()
        pltpu.make_async_copy(ef_hbm.at[pl.ds(0, SBC), :], ef_sb, sidx).wait()
        for b in range(2):
            _issue_gather(l_hbm, r_hbm, src_sb, dst_sb, b,
                          lrows[b], rrows[b], sg[b])

        def pair(it, sq_c):
            s, q = sq_c
            c = it * 2
            for b in range(2):
                cc = c + b
                _drain_gather(l_hbm, r_hbm, src_sb, dst_sb,
                              lrows[b], rrows[b], sg[b])

                def egroup(eg, sq_in, b=b, cc=cc):
                    s_, q_ = sq_in
                    e0 = eg * 16
                    ef16 = ef_sb[cc, pl.ds(e0, 16)]
                    for i in range(16):
                        efb = jnp.full((16,), ef16[i], jnp.float32)
                        for g in range(NG):
                            j = lrows[b][e0 + i, pl.ds(16 * g, 16)] \
                                + rrows[b][e0 + i, pl.ds(16 * g, 16)] \
                                + efb * wg[g]
                            s_ = s_[:g] + (s_[g] + j,) + s_[g + 1:]
                            q_ = q_[:g] + (q_[g] + j * j,) + q_[g + 1:]
                    return (s_, q_)

                s, q = lax.fori_loop(0, _NFULL, egroup, (s, q))
                if _TAIL:
                    e0 = CH - 16
                    ef16 = ef_sb[cc, pl.ds(e0, 16)]
                    for i in range(16 - _TAIL, 16):
                        efb = jnp.full((16,), ef16[i], jnp.float32)
                        for g in range(NG):
                            j = lrows[b][e0 + i, pl.ds(16 * g, 16)] \
                                + rrows[b][e0 + i, pl.ds(16 * g, 16)] \
                                + efb * wg[g]
                            s = s[:g] + (s[g] + j,) + s[g + 1:]
                            q = q[:g] + (q[g] + j * j,) + q[g + 1:]

                @pl.when(cc + 2 < SBC)
                def _():
                    _issue_gather(l_hbm, r_hbm, src_sb, dst_sb, cc + 2,
                                  lrows[b], rrows[b], sg[b])
            return (s, q)

        s, q = lax.fori_loop(0, SBC // 2, pair,
                             ((zero,) * NG, (zero,) * NG))
        for g in range(NG):
            sum_v[pl.ds(16 * g, 16)] += s[g]
            sq_v[pl.ds(16 * g, 16)] += q[g]
        return carry

    lax.fori_loop(0, nsb, superblock, 0)
    pltpu.sync_copy(sum_v, osum_hbm.at[wid])
    pltpu.sync_copy(sq_v, osq_hbm.at[wid])


def _sc_stats(L, R, src2, dst2, ef2, wvec):
    n_edges = src2.shape[0] * src2.shape[1]
    mesh = plsc.VectorSubcoreMesh(core_axis_name="c", subcore_axis_name="s")
    return pl.kernel(
        functools.partial(_sc_stats_body, n_edges),
        mesh=mesh,
        compiler_params=_SC_PARAMS,
        out_type=[jax.ShapeDtypeStruct((NW, EMB), jnp.float32)] * 2,
        scratch_types=[
            pltpu.VMEM((SBC, CH), jnp.int32),
            pltpu.VMEM((SBC, CH), jnp.int32),
            pltpu.VMEM((SBC, CH), jnp.float32),
            pltpu.VMEM((CH, EMB), jnp.float32),
            pltpu.VMEM((CH, EMB), jnp.float32),
            pltpu.VMEM((CH, EMB), jnp.float32),
            pltpu.VMEM((CH, EMB), jnp.float32),
            pltpu.VMEM((EMB,), jnp.float32),
            pltpu.VMEM((EMB,), jnp.float32),
            pltpu.VMEM((EMB,), jnp.float32),
            pltpu.SemaphoreType.DMA,
            pltpu.SemaphoreType.DMA,
            pltpu.SemaphoreType.DMA,
        ],
    )(L, R, src2, dst2, ef2, wvec)


def _sc_scatter_body(n_edges, n_right,
                     l_hbm, r_hbm, src_hbm, dst_hbm, ef_hbm, w_hbm,
                     scale_hbm, shift_hbm, out_hbm,
                     src_sb, dst_sb, ef_sb, l0, r0, l1, r1,
                     w_v, scale_v, shift_v, j0, j1,
                     acc_sh, sidx, sg0, sg1, ss0, ss1, zsem):
    cid = lax.axis_index("c")
    sid = lax.axis_index("s")
    wid = sid * NC + cid
    cpt = n_edges // NW // CH
    nsb = cpt // SBC
    row_base = wid * cpt
    nzch = n_right // ZROWS
    nzt = (nzch + NS - 1) // NS

    zero = jnp.zeros((16,), jnp.float32)
    onecol = jnp.where(lax.iota(jnp.int32, 16) == 0,
                       jnp.float32(1.0), jnp.float32(0.0))

    def zr(r, carry):
        for g in range(ACC_W // 16):
            j0[r, pl.ds(16 * g, 16)] = zero
            j1[r, pl.ds(16 * g, 16)] = zero
        return carry
    lax.fori_loop(0, CH, zr, 0)
    for t in range(nzt):
        k = sid + NS * t

        @pl.when(k < nzch)
        def _():
            rz = pl.multiple_of(k * ZROWS, 8)
            pltpu.async_copy(j0, acc_sh.at[pl.ds(rz, ZROWS), :], zsem)
    for t in range(nzt):
        k = sid + NS * t

        @pl.when(k < nzch)
        def _():
            pltpu.make_async_copy(
                j0, acc_sh.at[pl.ds(0, ZROWS), :], zsem).wait()
    plsc.subcore_barrier()

    def orow(r, carry):
        j0[r, pl.ds(EMB, 16)] = onecol
        j1[r, pl.ds(EMB, 16)] = onecol
        return carry
    lax.fori_loop(0, CH, orow, 0)

    pltpu.sync_copy(w_hbm, w_v)
    pltpu.sync_copy(scale_hbm, scale_v)
    pltpu.sync_copy(shift_hbm, shift_v)
    wg = [w_v[pl.ds(16 * g, 16)] for g in range(NG)]
    sg_ = [scale_v[pl.ds(16 * g, 16)] for g in range(NG)]
    tg = [shift_v[pl.ds(16 * g, 16)] for g in range(NG)]

    lrows = [l0, l1]
    rrows = [r0, r1]
    jbuf = [j0, j1]
    sg = [sg0, sg1]
    ss = [ss0, ss1]

    def superblock(sb, carry):
        r0_ = row_base + sb * SBC
        pltpu.async_copy(src_hbm.at[pl.ds(r0_, SBC), :], src_sb, sidx)
        pltpu.async_copy(dst_hbm.at[pl.ds(r0_, SBC), :], dst_sb, sidx)
        pltpu.async_copy(ef_hbm.at[pl.ds(r0_, SBC), :], ef_sb, sidx)
        pltpu.make_async_copy(src_hbm.at[pl.ds(0, SBC), :], src_sb, sidx).wait()
        pltpu.make_async_copy(dst_hbm.at[pl.ds(0, SBC), :], dst_sb, sidx).wait()
        pltpu.make_async_copy(ef_hbm.at[pl.ds(0, SBC), :], ef_sb, sidx).wait()
        for b in range(2):
            _issue_gather(l_hbm, r_hbm, src_sb, dst_sb, b,
                          lrows[b], rrows[b], sg[b])

        def pair(it, carry2):
            c = it * 2
            for b in range(2):
                cc = c + b
                _drain_gather(l_hbm, r_hbm, src_sb, dst_sb,
                              lrows[b], rrows[b], sg[b])

                @pl.when(cc >= 2)
                def _():
                    pltpu.make_async_copy(
                        jbuf[b], acc_sh.at[dst_sb.at[0]], ss[b]).wait()

                def egroup(eg, cz, b=b, cc=cc):
                    e0 = eg * 16
                    ef16 = ef_sb[cc, pl.ds(e0, 16)]
                    for i in range(16):
                        efb = jnp.full((16,), ef16[i], jnp.float32)
                        for g in range(NG):
                            x = lrows[b][e0 + i, pl.ds(16 * g, 16)] \
                                + rrows[b][e0 + i, pl.ds(16 * g, 16)] \
                                + efb * wg[g]
                            jbuf[b][e0 + i, pl.ds(16 * g, 16)] = jnp.maximum(
                                x * sg_[g] + tg[g], 0.0)
                    return cz

                lax.fori_loop(0, _NFULL, egroup, 0)
                if _TAIL:
                    e0 = CH - 16
                    ef16 = ef_sb[cc, pl.ds(e0, 16)]
                    for i in range(16 - _TAIL, 16):
                        efb = jnp.full((16,), ef16[i], jnp.float32)
                        for g in range(NG):
                            x = lrows[b][e0 + i, pl.ds(16 * g, 16)] \
                                + rrows[b][e0 + i, pl.ds(16 * g, 16)] \
                                + efb * wg[g]
                            jbuf[b][e0 + i, pl.ds(16 * g, 16)] = jnp.maximum(
                                x * sg_[g] + tg[g], 0.0)
                pltpu.async_copy(jbuf[b], acc_sh.at[dst_sb.at[cc]], ss[b],
                                 add=True)

                @pl.when(cc + 2 < SBC)
                def _():
                    _issue_gather(l_hbm, r_hbm, src_sb, dst_sb, cc + 2,
                                  lrows[b], rrows[b], sg[b])
            return carry2

        lax.fori_loop(0, SBC // 2, pair, 0)
        for b in range(2):
            pltpu.make_async_copy(jbuf[b], acc_sh.at[dst_sb.at[0]],
                                  ss[b]).wait()
        return carry

    lax.fori_loop(0, nsb, superblock, 0)
    plsc.subcore_barrier()

    for t in range(nzt):
        k = sid + NS * t

        @pl.when(k < nzch)
        def _():
            rz = pl.multiple_of(k * ZROWS, 8)
            pltpu.async_copy(acc_sh.at[pl.ds(rz, ZROWS), :],
                             out_hbm.at[cid, pl.ds(rz, ZROWS), :], zsem)
    for t in range(nzt):
        k = sid + NS * t

        @pl.when(k < nzch)
        def _():
            pltpu.make_async_copy(
                acc_sh.at[pl.ds(0, ZROWS), :],
                out_hbm.at[cid, pl.ds(0, ZROWS), :], zsem).wait()


def _sc_scatter(L, R, src2, dst2, ef2, wvec, scale, shift):
    n_edges = src2.shape[0] * src2.shape[1]
    n_right = R.shape[0]
    mesh = plsc.VectorSubcoreMesh(core_axis_name="c", subcore_axis_name="s")
    return pl.kernel(
        functools.partial(_sc_scatter_body, n_edges, n_right),
        mesh=mesh,
        compiler_params=_SC_PARAMS,
        out_type=jax.ShapeDtypeStruct((NC, n_right, ACC_W), jnp.float32),
        scratch_types=[
            pltpu.VMEM((SBC, CH), jnp.int32),
            pltpu.VMEM((SBC, CH), jnp.int32),
            pltpu.VMEM((SBC, CH), jnp.float32),
            pltpu.VMEM((CH, EMB), jnp.float32),
            pltpu.VMEM((CH, EMB), jnp.float32),
            pltpu.VMEM((CH, EMB), jnp.float32),
            pltpu.VMEM((CH, EMB), jnp.float32),
            pltpu.VMEM((EMB,), jnp.float32),
            pltpu.VMEM((EMB,), jnp.float32),
            pltpu.VMEM((EMB,), jnp.float32),
            pltpu.VMEM((CH, ACC_W), jnp.float32),
            pltpu.VMEM((CH, ACC_W), jnp.float32),
            pltpu.VMEM_SHARED((n_right, ACC_W), jnp.float32),
            pltpu.SemaphoreType.DMA,
            pltpu.SemaphoreType.DMA,
            pltpu.SemaphoreType.DMA,
            pltpu.SemaphoreType.DMA,
            pltpu.SemaphoreType.DMA,
            pltpu.SemaphoreType.DMA,
        ],
    )(L, R, src2, dst2, ef2, wvec, scale, shift)


def _tail_body(acc_ref, rf_ref, wf_ref, bf_ref, g2_ref, b2_ref,
               wo1a_ref, wo1b_ref, bo1_ref, wo2_ref, bo2_ref, out_ref):
    accs = acc_ref[0] + acc_ref[1]
    feat = accs[:, :EMB]
    cnt = accs[:, EMB:EMB + 1]
    conv = _dotT(feat, wf_ref[...]) + cnt * bf_ref[...]
    mu = jnp.mean(conv, axis=0, keepdims=True)
    var = jnp.mean((conv - mu) ** 2, axis=0, keepdims=True)
    convn = g2_ref[...] * (conv - mu) / jnp.sqrt(var + 1e-5) + b2_ref[...]
    h = jnp.maximum(
        _dotT(convn, wo1a_ref[...]) + _dotT(rf_ref[...], wo1b_ref[...])
        + bo1_ref[...], 0.0)
    out_ref[...] = jnp.maximum(_dotT(h, wo2_ref[...]) + bo2_ref[...], 0.0)


def _tc_tail(acc, rf, W_f, b_f, gamma2, beta2, W_o1, b_o1, W_o2, b_o2):
    n = rf.shape[0]
    full2 = pl.BlockSpec((EMB, EMB), lambda: (0, 0))
    row = pl.BlockSpec((1, EMB), lambda: (0, 0))
    return pl.pallas_call(
        _tail_body,
        in_specs=[
            pl.BlockSpec((NC, n, ACC_W), lambda: (0, 0, 0)),
            pl.BlockSpec((n, EMB), lambda: (0, 0)),
            full2, row, row, row, full2, full2, row, full2, row,
        ],
        out_specs=pl.BlockSpec((n, EMB), lambda: (0, 0)),
        out_shape=jax.ShapeDtypeStruct((n, EMB), jnp.float32),
    )(acc, rf, W_f, b_f.reshape(1, EMB), gamma2.reshape(1, EMB),
      beta2.reshape(1, EMB), W_o1[:, :EMB], W_o1[:, EMB:],
      b_o1.reshape(1, EMB), W_o2, b_o2.reshape(1, EMB))


def kernel(left_features, edge_indices, edge_features, right_features,
           scatter_out_size, W_l, b_l, W_e, W_r, gamma1, beta1,
           W_f, b_f, gamma2, beta2, W_o1, b_o1, W_o2, b_o2):
    n_edges = edge_indices.shape[1]
    src2 = edge_indices[0].astype(jnp.int32).reshape(n_edges // CH, CH)
    dst2 = edge_indices[1].astype(jnp.int32).reshape(n_edges // CH, CH)
    ef2 = edge_features[:, 0].astype(jnp.float32).reshape(n_edges // CH, CH)
    wvec = W_e[:, 0].astype(jnp.float32)

    L, R = _tc_lr(left_features, right_features, W_l, b_l, W_r)

    psum, psq = _sc_stats(L, R, src2, dst2, ef2, wvec)
    s1 = jnp.sum(psum, axis=0)
    s2 = jnp.sum(psq, axis=0)
    mu = s1 / n_edges
    var = s2 / n_edges - mu * mu
    inv = 1.0 / jnp.sqrt(var + 1e-5)
    scale = gamma1 * inv
    shift = beta1 - mu * scale

    acc = _sc_scatter(L, R, src2, dst2, ef2, wvec, scale, shift)

    return _tc_tail(acc, right_features, W_f, b_f, gamma2, beta2,
                    W_o1, b_o1, W_o2, b_o2)
